# Initial kernel scaffold; baseline (speedup 1.0000x reference)
import jax
import jax.numpy as jnp
from jax.experimental import pallas as pl


def kernel(x, edge_index, edge_attr, W1, b1, F1, bf1, W2, b2, F2, bf2, W3, b3, F3, bf3, Wl, bl):
    raise NotImplementedError("write your pallas kernel here")


# speedup vs baseline: 1.4361x; 1.4361x over previous
import functools

import jax
import jax.numpy as jnp
from jax import lax
from jax.experimental import pallas as pl
from jax.experimental.pallas import tpu as pltpu
from jax.experimental.pallas import tpu_sc as plsc

N_NODES = 10000
N_EDGES = 320000
DPAD = 128
Z = 96
W_TAB = DPAD + Z
NC, NS = 2, 16
NW = NC * NS
EPW = N_EDGES // NW
KB = 80
NBLK = EPW // KB
RPS = N_NODES // NS

BN = 2000
BE = 2000


def _tables_body(x_ref, wa_ref, wb_ref, s_ref, d_ref):
    x = x_ref[...]
    q = jnp.dot(x, wb_ref[...], preferred_element_type=jnp.float32)
    p = jnp.dot(x, wa_ref[...] - wb_ref[...],
                preferred_element_type=jnp.float32)
    dl = x.shape[1]
    if dl < DPAD:
        pad = jnp.zeros((x.shape[0], DPAD - dl), jnp.float32)
        s_ref[...] = jnp.concatenate([x, pad, q], axis=1)
        d_ref[...] = jnp.concatenate([x, pad, p], axis=1)
    else:
        s_ref[...] = jnp.concatenate([x, q], axis=1)
        d_ref[...] = jnp.concatenate([x, p], axis=1)


def _make_tables(x, wa, wb):
    dl = x.shape[1]
    return pl.pallas_call(
        _tables_body,
        grid=(N_NODES // BN,),
        in_specs=[
            pl.BlockSpec((BN, dl), lambda i: (i, 0)),
            pl.BlockSpec((dl, Z), lambda i: (0, 0)),
            pl.BlockSpec((dl, Z), lambda i: (0, 0)),
        ],
        out_specs=[pl.BlockSpec((BN, W_TAB), lambda i: (i, 0))] * 2,
        out_shape=[jax.ShapeDtypeStruct((N_NODES, W_TAB), jnp.float32)] * 2,
    )(x, wa, wb)


def _gather_body(ts, td, src, dst, outs, outd,
                 sidx, didx, srows, drows, sem_s, sem_d):
    wid = lax.axis_index("s") * NC + lax.axis_index("c")

    def blk(i, carry):
        base = wid * EPW + i * KB
        pltpu.sync_copy(src.at[pl.ds(base, KB)], sidx)
        pltpu.sync_copy(dst.at[pl.ds(base, KB)], didx)
        cp1 = pltpu.async_copy(ts.at[sidx], srows, sem_s)
        cp2 = pltpu.async_copy(td.at[didx], drows, sem_d)
        cp1.wait()
        cp2.wait()
        pltpu.sync_copy(srows, outs.at[pl.ds(base, KB)])
        pltpu.sync_copy(drows, outd.at[pl.ds(base, KB)])
        return carry

    lax.fori_loop(0, NBLK, blk, 0)


def _sc_mesh():
    return plsc.VectorSubcoreMesh(core_axis_name="c", subcore_axis_name="s",
                                  num_cores=NC, num_subcores=NS)


@functools.cache
def _gather_call():
    return pl.kernel(
        _gather_body,
        out_type=[jax.ShapeDtypeStruct((N_EDGES, W_TAB), jnp.float32)] * 2,
        mesh=_sc_mesh(),
        compiler_params=pltpu.CompilerParams(use_tc_tiling_on_sc=False),
        scratch_types=[
            pltpu.VMEM((KB,), jnp.int32),
            pltpu.VMEM((KB,), jnp.int32),
            pltpu.VMEM((KB, W_TAB), jnp.float32),
            pltpu.VMEM((KB, W_TAB), jnp.float32),
            pltpu.SemaphoreType.DMA,
            pltpu.SemaphoreType.DMA,
        ],
    )


def _msg_body(gs_ref, gd_ref, ea_ref, wc_ref, wd_ref, b_ref, out_ref):
    gs = gs_ref[...]
    gd = gd_ref[...]
    xs = gs[:, :DPAD]
    xd = gd[:, :DPAD]
    qs = gs[:, DPAD:]
    pd = gd[:, DPAD:]
    acc = jnp.dot(xs * xd, wc_ref[...], preferred_element_type=jnp.float32)
    acc = acc + jnp.dot(ea_ref[...], wd_ref[...],
                        preferred_element_type=jnp.float32)
    out_ref[...] = jnp.maximum(acc + qs + pd + b_ref[...], 0.0)


def _make_msg(gs, gd, ea, wc_pad, wd, b_row):
    ea_w = ea.shape[1]
    return pl.pallas_call(
        _msg_body,
        grid=(N_EDGES // BE,),
        in_specs=[
            pl.BlockSpec((BE, W_TAB), lambda i: (i, 0)),
            pl.BlockSpec((BE, W_TAB), lambda i: (i, 0)),
            pl.BlockSpec((BE, ea_w), lambda i: (i, 0)),
            pl.BlockSpec((DPAD, Z), lambda i: (0, 0)),
            pl.BlockSpec((ea_w, Z), lambda i: (0, 0)),
            pl.BlockSpec((1, Z), lambda i: (0, 0)),
        ],
        out_specs=pl.BlockSpec((BE, Z), lambda i: (i, 0)),
        out_shape=jax.ShapeDtypeStruct((N_EDGES, Z), jnp.float32),
    )(gs, gd, ea, wc_pad, wd, b_row)


def _scatter_deg_body(msg, dstidx, zz, ones8, z8, num_out, deg_out,
                      idxv, rowsv, onesv, accum, accum8):
    c = lax.axis_index("c")
    s = lax.axis_index("s")
    wid = s * NC + c
    r0 = s * RPS
    pltpu.sync_copy(zz.at[pl.ds(r0, RPS)], accum.at[pl.ds(r0, RPS)])
    pltpu.sync_copy(z8.at[pl.ds(r0, RPS)], accum8.at[pl.ds(r0, RPS)])
    pltpu.sync_copy(ones8, onesv)
    plsc.subcore_barrier()

    def blk(i, carry):
        base = wid * EPW + i * KB
        pltpu.sync_copy(dstidx.at[pl.ds(base, KB)], idxv)
        pltpu.sync_copy(msg.at[pl.ds(base, KB)], rowsv)
        pltpu.sync_copy(rowsv, accum.at[idxv], add=True)
        pltpu.sync_copy(onesv, accum8.at[idxv], add=True)
        return carry

    lax.fori_loop(0, NBLK, blk, 0)
    plsc.subcore_barrier()
    pltpu.sync_copy(accum.at[pl.ds(r0, RPS)], num_out.at[c, pl.ds(r0, RPS)])
    pltpu.sync_copy(accum8.at[pl.ds(r0, RPS)], deg_out.at[c, pl.ds(r0, RPS)])


@functools.cache
def _scatter_deg_call():
    return pl.kernel(
        _scatter_deg_body,
        out_type=[
            jax.ShapeDtypeStruct((NC, N_NODES, Z), jnp.float32),
            jax.ShapeDtypeStruct((NC, N_NODES, 8), jnp.float32),
        ],
        mesh=_sc_mesh(),
        compiler_params=pltpu.CompilerParams(use_tc_tiling_on_sc=False),
        scratch_types=[
            pltpu.VMEM((KB,), jnp.int32),
            pltpu.VMEM((KB, Z), jnp.float32),
            pltpu.VMEM((KB, 8), jnp.float32),
            pltpu.VMEM_SHARED((N_NODES, Z), jnp.float32),
            pltpu.VMEM_SHARED((N_NODES, 8), jnp.float32),
        ],
    )


def _scatter_body(msg, dstidx, zz, num_out, idxv, rowsv, accum):
    c = lax.axis_index("c")
    s = lax.axis_index("s")
    wid = s * NC + c
    r0 = s * RPS
    pltpu.sync_copy(zz.at[pl.ds(r0, RPS)], accum.at[pl.ds(r0, RPS)])
    plsc.subcore_barrier()

    def blk(i, carry):
        base = wid * EPW + i * KB
        pltpu.sync_copy(dstidx.at[pl.ds(base, KB)], idxv)
        pltpu.sync_copy(msg.at[pl.ds(base, KB)], rowsv)
        pltpu.sync_copy(rowsv, accum.at[idxv], add=True)
        return carry

    lax.fori_loop(0, NBLK, blk, 0)
    plsc.subcore_barrier()
    pltpu.sync_copy(accum.at[pl.ds(r0, RPS)], num_out.at[c, pl.ds(r0, RPS)])


@functools.cache
def _scatter_call():
    return pl.kernel(
        _scatter_body,
        out_type=jax.ShapeDtypeStruct((NC, N_NODES, Z), jnp.float32),
        mesh=_sc_mesh(),
        compiler_params=pltpu.CompilerParams(use_tc_tiling_on_sc=False),
        scratch_types=[
            pltpu.VMEM((KB,), jnp.int32),
            pltpu.VMEM((KB, Z), jnp.float32),
            pltpu.VMEM_SHARED((N_NODES, Z), jnp.float32),
        ],
    )


def _recip_body(d0_ref, d1_ref, out_ref):
    deg = d0_ref[:, :1] + d1_ref[:, :1]
    out_ref[...] = 1.0 / jnp.maximum(deg, 1.0)


def _make_recip(d0, d1):
    return pl.pallas_call(
        _recip_body,
        grid=(N_NODES // BN,),
        in_specs=[pl.BlockSpec((BN, 8), lambda i: (i, 0))] * 2,
        out_specs=pl.BlockSpec((BN, 1), lambda i: (i, 0)),
        out_shape=jax.ShapeDtypeStruct((N_NODES, 1), jnp.float32),
    )(d0, d1)


def _update_body(x_ref, n0_ref, n1_ref, r_ref, fa_ref, fb_ref, bf_ref,
                 out_ref):
    agg = (n0_ref[...] + n1_ref[...]) * r_ref[...]
    acc = jnp.dot(x_ref[...], fa_ref[...], preferred_element_type=jnp.float32)
    acc = acc + jnp.dot(agg, fb_ref[...], preferred_element_type=jnp.float32)
    out_ref[...] = jnp.maximum(acc + bf_ref[...], 0.0)


def _make_update(x, n0, n1, recip, fa, fb, bf_row):
    dl = x.shape[1]
    return pl.pallas_call(
        _update_body,
        grid=(N_NODES // BN,),
        in_specs=[
            pl.BlockSpec((BN, dl), lambda i: (i, 0)),
            pl.BlockSpec((BN, Z), lambda i: (i, 0)),
            pl.BlockSpec((BN, Z), lambda i: (i, 0)),
            pl.BlockSpec((BN, 1), lambda i: (i, 0)),
            pl.BlockSpec((dl, Z), lambda i: (0, 0)),
            pl.BlockSpec((Z, Z), lambda i: (0, 0)),
            pl.BlockSpec((1, Z), lambda i: (0, 0)),
        ],
        out_specs=pl.BlockSpec((BN, Z), lambda i: (i, 0)),
        out_shape=jax.ShapeDtypeStruct((N_NODES, Z), jnp.float32),
    )(x, n0, n1, recip, fa, fb, bf_row)


def _final_body(x1_ref, x2_ref, x3_ref, w1_ref, w2_ref, w3_ref, b_ref,
                out_ref):
    acc = jnp.dot(x1_ref[...], w1_ref[...], preferred_element_type=jnp.float32)
    acc = acc + jnp.dot(x2_ref[...], w2_ref[...],
                        preferred_element_type=jnp.float32)
    acc = acc + jnp.dot(x3_ref[...], w3_ref[...],
                        preferred_element_type=jnp.float32)
    out_ref[...] = jnp.maximum(acc + b_ref[...], 0.0)


def _make_final(x1, x2, x3, w1, w2, w3, b_row):
    return pl.pallas_call(
        _final_body,
        grid=(N_NODES // BN,),
        in_specs=[pl.BlockSpec((BN, Z), lambda i: (i, 0))] * 3
        + [pl.BlockSpec((Z, Z), lambda i: (0, 0))] * 3
        + [pl.BlockSpec((1, Z), lambda i: (0, 0))],
        out_specs=pl.BlockSpec((BN, Z), lambda i: (i, 0)),
        out_shape=jax.ShapeDtypeStruct((N_NODES, Z), jnp.float32),
    )(x1, x2, x3, w1, w2, w3, b_row)


def kernel(x, edge_index, edge_attr, W1, b1, F1, bf1, W2, b2, F2, bf2,
           W3, b3, F3, bf3, Wl, bl):
    src = edge_index[0].astype(jnp.int32)
    dst = edge_index[1].astype(jnp.int32)
    zeros_z = jnp.zeros((N_NODES, Z), jnp.float32)
    zeros_8 = jnp.zeros((N_NODES, 8), jnp.float32)
    ones_8 = jnp.ones((KB, 8), jnp.float32)

    recip = None

    def layer(xc, wfull, b, fw, bf, first):
        nonlocal recip
        dl = xc.shape[1]
        wa = wfull[:dl]
        wb = wfull[dl:2 * dl]
        wc = wfull[2 * dl:3 * dl]
        wd = wfull[3 * dl:]
        if dl < DPAD:
            wc = jnp.pad(wc, ((0, DPAD - dl), (0, 0)))
        s_tab, d_tab = _make_tables(xc, wa, wb)
        gs, gd = _gather_call()(s_tab, d_tab, src, dst)
        msg = _make_msg(gs, gd, edge_attr, wc, wd, b.reshape(1, Z))
        if first:
            num, degp = _scatter_deg_call()(msg, dst, zeros_z, ones_8,
                                            zeros_8)
            recip = _make_recip(degp[0], degp[1])
        else:
            num = _scatter_call()(msg, dst, zeros_z)
        return _make_update(xc, num[0], num[1], recip, fw[:dl], fw[dl:],
                            bf.reshape(1, Z))

    x1 = layer(x, W1, b1, F1, bf1, True)
    x2 = layer(x1, W2, b2, F2, bf2, False)
    x3 = layer(x2, W3, b3, F3, bf3, False)
    return _make_final(x1, x2, x3, Wl[:Z], Wl[Z:2 * Z], Wl[2 * Z:],
                       bl.reshape(1, Z))


# speedup vs baseline: 2.3618x; 1.6446x over previous
import functools

import jax
import jax.numpy as jnp
from jax import lax
from jax.experimental import pallas as pl
from jax.experimental.pallas import tpu as pltpu
from jax.experimental.pallas import tpu_sc as plsc

N_NODES = 10000
N_EDGES = 320000
DPAD = 128
Z = 96
W_TAB = 256
MSGW = 128
NC, NS = 2, 16
NW = NC * NS
EPW = N_EDGES // NW
KB = 80
NBLK = EPW // KB
RPS = 624
RTAIL = N_NODES - NS * RPS

BN = 2000
BE = 2000


def _tables_body(x_ref, wa_ref, wb_ref, s_ref, d_ref):
    x = x_ref[...]
    q = jnp.dot(x, wb_ref[...], preferred_element_type=jnp.float32)
    p = jnp.dot(x, wa_ref[...] - wb_ref[...],
                preferred_element_type=jnp.float32)
    dl = x.shape[1]
    zpad = jnp.zeros((x.shape[0], W_TAB - DPAD - Z), jnp.float32)
    if dl < DPAD:
        xpad = jnp.zeros((x.shape[0], DPAD - dl), jnp.float32)
        s_ref[...] = jnp.concatenate([x, xpad, q, zpad], axis=1)
        d_ref[...] = jnp.concatenate([x, xpad, p, zpad], axis=1)
    else:
        s_ref[...] = jnp.concatenate([x, q, zpad], axis=1)
        d_ref[...] = jnp.concatenate([x, p, zpad], axis=1)


def _make_tables(x, wa, wb):
    dl = x.shape[1]
    return pl.pallas_call(
        _tables_body,
        grid=(N_NODES // BN,),
        in_specs=[
            pl.BlockSpec((BN, dl), lambda i: (i, 0)),
            pl.BlockSpec((dl, Z), lambda i: (0, 0)),
            pl.BlockSpec((dl, Z), lambda i: (0, 0)),
        ],
        out_specs=[pl.BlockSpec((BN, W_TAB), lambda i: (i, 0))] * 2,
        out_shape=[jax.ShapeDtypeStruct((N_NODES, W_TAB), jnp.float32)] * 2,
    )(x, wa, wb)


def _gather_body(ts, td, src, dst, outs, outd,
                 sidx, didx, srows, drows, sem_s, sem_d):
    wid = lax.axis_index("s") * NC + lax.axis_index("c")

    def blk(i, carry):
        base = wid * EPW + i * KB
        pltpu.sync_copy(src.at[pl.ds(base, KB)], sidx)
        pltpu.sync_copy(dst.at[pl.ds(base, KB)], didx)
        cp1 = pltpu.async_copy(ts.at[sidx], srows, sem_s)
        cp2 = pltpu.async_copy(td.at[didx], drows, sem_d)
        cp1.wait()
        cp2.wait()
        pltpu.sync_copy(srows, outs.at[pl.ds(base, KB)])
        pltpu.sync_copy(drows, outd.at[pl.ds(base, KB)])
        return carry

    lax.fori_loop(0, NBLK, blk, 0)


def _sc_mesh():
    return plsc.VectorSubcoreMesh(core_axis_name="c", subcore_axis_name="s",
                                  num_cores=NC, num_subcores=NS)


@functools.cache
def _gather_call():
    return pl.kernel(
        _gather_body,
        out_type=[jax.ShapeDtypeStruct((N_EDGES, W_TAB), jnp.float32)] * 2,
        mesh=_sc_mesh(),
        scratch_types=[
            pltpu.VMEM((KB,), jnp.int32),
            pltpu.VMEM((KB,), jnp.int32),
            pltpu.VMEM((KB, W_TAB), jnp.float32),
            pltpu.VMEM((KB, W_TAB), jnp.float32),
            pltpu.SemaphoreType.DMA,
            pltpu.SemaphoreType.DMA,
        ],
    )


def _msg_body(gs_ref, gd_ref, ea_ref, wc_ref, wd_ref, b_ref, out_ref):
    gs = gs_ref[...]
    gd = gd_ref[...]
    xs = gs[:, :DPAD]
    xd = gd[:, :DPAD]
    qs = gs[:, DPAD:DPAD + Z]
    pd = gd[:, DPAD:DPAD + Z]
    acc = jnp.dot(xs * xd, wc_ref[...], preferred_element_type=jnp.float32)
    acc = acc + jnp.dot(ea_ref[...], wd_ref[...],
                        preferred_element_type=jnp.float32)
    msg = jnp.maximum(acc + qs + pd + b_ref[...], 0.0)
    n = msg.shape[0]
    one = jnp.ones((n, 1), jnp.float32)
    zpad = jnp.zeros((n, MSGW - Z - 1), jnp.float32)
    out_ref[...] = jnp.concatenate([msg, one, zpad], axis=1)


def _make_msg(gs, gd, ea, wc_pad, wd, b_row):
    ea_w = ea.shape[1]
    return pl.pallas_call(
        _msg_body,
        grid=(N_EDGES // BE,),
        in_specs=[
            pl.BlockSpec((BE, W_TAB), lambda i: (i, 0)),
            pl.BlockSpec((BE, W_TAB), lambda i: (i, 0)),
            pl.BlockSpec((BE, ea_w), lambda i: (i, 0)),
            pl.BlockSpec((DPAD, Z), lambda i: (0, 0)),
            pl.BlockSpec((ea_w, Z), lambda i: (0, 0)),
            pl.BlockSpec((1, Z), lambda i: (0, 0)),
        ],
        out_specs=pl.BlockSpec((BE, MSGW), lambda i: (i, 0)),
        out_shape=jax.ShapeDtypeStruct((N_EDGES, MSGW), jnp.float32),
    )(gs, gd, ea, wc_pad, wd, b_row)


def _scatter_body(msg, dstidx, zz, num_out, idxv, rowsv, accum):
    c = lax.axis_index("c")
    s = lax.axis_index("s")
    wid = s * NC + c
    r0 = s * RPS
    pltpu.sync_copy(zz.at[pl.ds(r0, RPS)], accum.at[pl.ds(r0, RPS)])

    @pl.when(s == 0)
    def _():
        pltpu.sync_copy(zz.at[pl.ds(NS * RPS, RTAIL)],
                        accum.at[pl.ds(NS * RPS, RTAIL)])

    plsc.subcore_barrier()

    def blk(i, carry):
        base = wid * EPW + i * KB
        pltpu.sync_copy(dstidx.at[pl.ds(base, KB)], idxv)
        pltpu.sync_copy(msg.at[pl.ds(base, KB)], rowsv)
        pltpu.sync_copy(rowsv, accum.at[idxv], add=True)
        return carry

    lax.fori_loop(0, NBLK, blk, 0)
    plsc.subcore_barrier()
    pltpu.sync_copy(accum.at[pl.ds(r0, RPS)], num_out.at[c, pl.ds(r0, RPS)])

    @pl.when(s == 0)
    def _():
        pltpu.sync_copy(accum.at[pl.ds(NS * RPS, RTAIL)],
                        num_out.at[c, pl.ds(NS * RPS, RTAIL)])


@functools.cache
def _scatter_call():
    return pl.kernel(
        _scatter_body,
        out_type=jax.ShapeDtypeStruct((NC, N_NODES, MSGW), jnp.float32),
        mesh=_sc_mesh(),
        scratch_types=[
            pltpu.VMEM((KB,), jnp.int32),
            pltpu.VMEM((KB, MSGW), jnp.float32),
            pltpu.VMEM_SHARED((N_NODES, MSGW), jnp.float32),
        ],
    )


def _update_body(x_ref, n0_ref, n1_ref, fa_ref, fb_ref, bf_ref, out_ref):
    n0 = n0_ref[...]
    n1 = n1_ref[...]
    num = n0[:, :Z] + n1[:, :Z]
    deg = n0[:, Z:Z + 1] + n1[:, Z:Z + 1]
    agg = num / jnp.maximum(deg, 1.0)
    acc = jnp.dot(x_ref[...], fa_ref[...], preferred_element_type=jnp.float32)
    acc = acc + jnp.dot(agg, fb_ref[...], preferred_element_type=jnp.float32)
    out_ref[...] = jnp.maximum(acc + bf_ref[...], 0.0)


def _make_update(x, n0, n1, fa, fb, bf_row):
    dl = x.shape[1]
    return pl.pallas_call(
        _update_body,
        grid=(N_NODES // BN,),
        in_specs=[
            pl.BlockSpec((BN, dl), lambda i: (i, 0)),
            pl.BlockSpec((BN, MSGW), lambda i: (i, 0)),
            pl.BlockSpec((BN, MSGW), lambda i: (i, 0)),
            pl.BlockSpec((dl, Z), lambda i: (0, 0)),
            pl.BlockSpec((Z, Z), lambda i: (0, 0)),
            pl.BlockSpec((1, Z), lambda i: (0, 0)),
        ],
        out_specs=pl.BlockSpec((BN, Z), lambda i: (i, 0)),
        out_shape=jax.ShapeDtypeStruct((N_NODES, Z), jnp.float32),
    )(x, n0, n1, fa, fb, bf_row)


def _final_body(x1_ref, x2_ref, x3_ref, w1_ref, w2_ref, w3_ref, b_ref,
                out_ref):
    acc = jnp.dot(x1_ref[...], w1_ref[...], preferred_element_type=jnp.float32)
    acc = acc + jnp.dot(x2_ref[...], w2_ref[...],
                        preferred_element_type=jnp.float32)
    acc = acc + jnp.dot(x3_ref[...], w3_ref[...],
                        preferred_element_type=jnp.float32)
    out_ref[...] = jnp.maximum(acc + b_ref[...], 0.0)


def _make_final(x1, x2, x3, w1, w2, w3, b_row):
    return pl.pallas_call(
        _final_body,
        grid=(N_NODES // BN,),
        in_specs=[pl.BlockSpec((BN, Z), lambda i: (i, 0))] * 3
        + [pl.BlockSpec((Z, Z), lambda i: (0, 0))] * 3
        + [pl.BlockSpec((1, Z), lambda i: (0, 0))],
        out_specs=pl.BlockSpec((BN, Z), lambda i: (i, 0)),
        out_shape=jax.ShapeDtypeStruct((N_NODES, Z), jnp.float32),
    )(x1, x2, x3, w1, w2, w3, b_row)


def kernel(x, edge_index, edge_attr, W1, b1, F1, bf1, W2, b2, F2, bf2,
           W3, b3, F3, bf3, Wl, bl):
    src = edge_index[0].astype(jnp.int32)
    dst = edge_index[1].astype(jnp.int32)
    zeros_m = jnp.zeros((N_NODES, MSGW), jnp.float32)

    def layer(xc, wfull, b, fw, bf):
        dl = xc.shape[1]
        wa = wfull[:dl]
        wb = wfull[dl:2 * dl]
        wc = wfull[2 * dl:3 * dl]
        wd = wfull[3 * dl:]
        if dl < DPAD:
            wc = jnp.pad(wc, ((0, DPAD - dl), (0, 0)))
        s_tab, d_tab = _make_tables(xc, wa, wb)
        gs, gd = _gather_call()(s_tab, d_tab, src, dst)
        msg = _make_msg(gs, gd, edge_attr, wc, wd, b.reshape(1, Z))
        num = _scatter_call()(msg, dst, zeros_m)
        return _make_update(xc, num[0], num[1], fw[:dl], fw[dl:],
                            bf.reshape(1, Z))

    x1 = layer(x, W1, b1, F1, bf1)
    x2 = layer(x1, W2, b2, F2, bf2)
    x3 = layer(x2, W3, b3, F3, bf3)
    return _make_final(x1, x2, x3, Wl[:Z], Wl[Z:2 * Z], Wl[2 * Z:],
                       bl.reshape(1, Z))


# speedup vs baseline: 3.0244x; 1.2805x over previous
import functools

import jax
import jax.numpy as jnp
from jax import lax
from jax.experimental import pallas as pl
from jax.experimental.pallas import tpu as pltpu
from jax.experimental.pallas import tpu_sc as plsc

N_NODES = 10000
N_EDGES = 320000
DPAD = 128
Z = 96
W_TAB = 128
MSGW = 128
NC, NS = 2, 16
NW = NC * NS
EPW = N_EDGES // NW
KB = 80
NBLK = EPW // KB
RPS = 624
RTAIL = N_NODES - NS * RPS

BN = 2000
BE = 2000


def _bf16_bits(v):
    u = jax.lax.bitcast_convert_type(v, jnp.uint32)
    return (u + jnp.uint32(0x7FFF) + ((u >> 16) & jnp.uint32(1))) >> 16


def _pack2(first, second):
    return (_bf16_bits(second) << 16) | _bf16_bits(first)


def _unpack2(u):
    first = jax.lax.bitcast_convert_type(u << 16, jnp.float32)
    second = jax.lax.bitcast_convert_type(u & jnp.uint32(0xFFFF0000),
                                          jnp.float32)
    return first, second


def _tables_body(x_ref, wa_ref, wb_ref, s_ref, d_ref):
    x = x_ref[...]
    q = jnp.dot(x, wb_ref[...], preferred_element_type=jnp.float32)
    p = jnp.dot(x, wa_ref[...] - wb_ref[...],
                preferred_element_type=jnp.float32)
    dl = x.shape[1]
    hpad = jnp.zeros((x.shape[0], DPAD - Z), jnp.float32)
    if dl < DPAD:
        xpad = jnp.zeros((x.shape[0], DPAD - dl), jnp.float32)
        x = jnp.concatenate([x, xpad], axis=1)
    qh = jnp.concatenate([q, hpad], axis=1)
    ph = jnp.concatenate([p, hpad], axis=1)
    s_ref[...] = _pack2(x, qh)
    d_ref[...] = _pack2(x, ph)


def _make_tables(x, wa, wb):
    dl = x.shape[1]
    return pl.pallas_call(
        _tables_body,
        grid=(N_NODES // BN,),
        in_specs=[
            pl.BlockSpec((BN, dl), lambda i: (i, 0)),
            pl.BlockSpec((dl, Z), lambda i: (0, 0)),
            pl.BlockSpec((dl, Z), lambda i: (0, 0)),
        ],
        out_specs=[pl.BlockSpec((BN, W_TAB), lambda i: (i, 0))] * 2,
        out_shape=[jax.ShapeDtypeStruct((N_NODES, W_TAB), jnp.uint32)] * 2,
    )(x, wa, wb)


def _gather_body(ts, td, src, dst, outs, outd,
                 sidx, didx, srows, drows, sem_s, sem_d):
    wid = lax.axis_index("s") * NC + lax.axis_index("c")

    def blk(i, carry):
        base = wid * EPW + i * KB
        pltpu.sync_copy(src.at[pl.ds(base, KB)], sidx)
        pltpu.sync_copy(dst.at[pl.ds(base, KB)], didx)
        cp1 = pltpu.async_copy(ts.at[sidx], srows, sem_s)
        cp2 = pltpu.async_copy(td.at[didx], drows, sem_d)
        cp1.wait()
        cp2.wait()
        pltpu.sync_copy(srows, outs.at[pl.ds(base, KB)])
        pltpu.sync_copy(drows, outd.at[pl.ds(base, KB)])
        return carry

    lax.fori_loop(0, NBLK, blk, 0)


def _sc_mesh():
    return plsc.VectorSubcoreMesh(core_axis_name="c", subcore_axis_name="s",
                                  num_cores=NC, num_subcores=NS)


@functools.cache
def _gather_call():
    return pl.kernel(
        _gather_body,
        out_type=[jax.ShapeDtypeStruct((N_EDGES, W_TAB), jnp.uint32)] * 2,
        mesh=_sc_mesh(),
        scratch_types=[
            pltpu.VMEM((KB,), jnp.int32),
            pltpu.VMEM((KB,), jnp.int32),
            pltpu.VMEM((KB, W_TAB), jnp.uint32),
            pltpu.VMEM((KB, W_TAB), jnp.uint32),
            pltpu.SemaphoreType.DMA,
            pltpu.SemaphoreType.DMA,
        ],
    )


def _msg_body(gs_ref, gd_ref, ea_ref, wc_ref, wd_ref, b_ref, out_ref):
    xs, qsh = _unpack2(gs_ref[...])
    xd, pdh = _unpack2(gd_ref[...])
    qs = qsh[:, :Z]
    pd = pdh[:, :Z]
    acc = jnp.dot(xs * xd, wc_ref[...], preferred_element_type=jnp.float32)
    acc = acc + jnp.dot(ea_ref[...], wd_ref[...],
                        preferred_element_type=jnp.float32)
    msg = jnp.maximum(acc + qs + pd + b_ref[...], 0.0)
    n = msg.shape[0]
    one = jnp.ones((n, 1), jnp.float32)
    zpad = jnp.zeros((n, MSGW - Z - 1), jnp.float32)
    out_ref[...] = jnp.concatenate([msg, one, zpad], axis=1)


def _make_msg(gs, gd, ea, wc_pad, wd, b_row):
    ea_w = ea.shape[1]
    return pl.pallas_call(
        _msg_body,
        grid=(N_EDGES // BE,),
        in_specs=[
            pl.BlockSpec((BE, W_TAB), lambda i: (i, 0)),
            pl.BlockSpec((BE, W_TAB), lambda i: (i, 0)),
            pl.BlockSpec((BE, ea_w), lambda i: (i, 0)),
            pl.BlockSpec((DPAD, Z), lambda i: (0, 0)),
            pl.BlockSpec((ea_w, Z), lambda i: (0, 0)),
            pl.BlockSpec((1, Z), lambda i: (0, 0)),
        ],
        out_specs=pl.BlockSpec((BE, MSGW), lambda i: (i, 0)),
        out_shape=jax.ShapeDtypeStruct((N_EDGES, MSGW), jnp.float32),
    )(gs, gd, ea, wc_pad, wd, b_row)


def _scatter_body(msg, dstidx, zz, num_out, idxv, rowsv, accum):
    c = lax.axis_index("c")
    s = lax.axis_index("s")
    wid = s * NC + c
    r0 = s * RPS
    pltpu.sync_copy(zz.at[pl.ds(r0, RPS)], accum.at[pl.ds(r0, RPS)])

    @pl.when(s == 0)
    def _():
        pltpu.sync_copy(zz.at[pl.ds(NS * RPS, RTAIL)],
                        accum.at[pl.ds(NS * RPS, RTAIL)])

    plsc.subcore_barrier()

    def blk(i, carry):
        base = wid * EPW + i * KB
        pltpu.sync_copy(dstidx.at[pl.ds(base, KB)], idxv)
        pltpu.sync_copy(msg.at[pl.ds(base, KB)], rowsv)
        pltpu.sync_copy(rowsv, accum.at[idxv], add=True)
        return carry

    lax.fori_loop(0, NBLK, blk, 0)
    plsc.subcore_barrier()
    pltpu.sync_copy(accum.at[pl.ds(r0, RPS)], num_out.at[c, pl.ds(r0, RPS)])

    @pl.when(s == 0)
    def _():
        pltpu.sync_copy(accum.at[pl.ds(NS * RPS, RTAIL)],
                        num_out.at[c, pl.ds(NS * RPS, RTAIL)])


@functools.cache
def _scatter_call():
    return pl.kernel(
        _scatter_body,
        out_type=jax.ShapeDtypeStruct((NC, N_NODES, MSGW), jnp.float32),
        mesh=_sc_mesh(),
        scratch_types=[
            pltpu.VMEM((KB,), jnp.int32),
            pltpu.VMEM((KB, MSGW), jnp.float32),
            pltpu.VMEM_SHARED((N_NODES, MSGW), jnp.float32),
        ],
    )


def _update_body(x_ref, n0_ref, n1_ref, fa_ref, fb_ref, bf_ref, out_ref):
    n0 = n0_ref[...]
    n1 = n1_ref[...]
    num = n0[:, :Z] + n1[:, :Z]
    deg = n0[:, Z:Z + 1] + n1[:, Z:Z + 1]
    agg = num / jnp.maximum(deg, 1.0)
    acc = jnp.dot(x_ref[...], fa_ref[...], preferred_element_type=jnp.float32)
    acc = acc + jnp.dot(agg, fb_ref[...], preferred_element_type=jnp.float32)
    out_ref[...] = jnp.maximum(acc + bf_ref[...], 0.0)


def _make_update(x, n0, n1, fa, fb, bf_row):
    dl = x.shape[1]
    return pl.pallas_call(
        _update_body,
        grid=(N_NODES // BN,),
        in_specs=[
            pl.BlockSpec((BN, dl), lambda i: (i, 0)),
            pl.BlockSpec((BN, MSGW), lambda i: (i, 0)),
            pl.BlockSpec((BN, MSGW), lambda i: (i, 0)),
            pl.BlockSpec((dl, Z), lambda i: (0, 0)),
            pl.BlockSpec((Z, Z), lambda i: (0, 0)),
            pl.BlockSpec((1, Z), lambda i: (0, 0)),
        ],
        out_specs=pl.BlockSpec((BN, Z), lambda i: (i, 0)),
        out_shape=jax.ShapeDtypeStruct((N_NODES, Z), jnp.float32),
    )(x, n0, n1, fa, fb, bf_row)


def _final_body(x1_ref, x2_ref, x3_ref, w1_ref, w2_ref, w3_ref, b_ref,
                out_ref):
    acc = jnp.dot(x1_ref[...], w1_ref[...], preferred_element_type=jnp.float32)
    acc = acc + jnp.dot(x2_ref[...], w2_ref[...],
                        preferred_element_type=jnp.float32)
    acc = acc + jnp.dot(x3_ref[...], w3_ref[...],
                        preferred_element_type=jnp.float32)
    out_ref[...] = jnp.maximum(acc + b_ref[...], 0.0)


def _make_final(x1, x2, x3, w1, w2, w3, b_row):
    return pl.pallas_call(
        _final_body,
        grid=(N_NODES // BN,),
        in_specs=[pl.BlockSpec((BN, Z), lambda i: (i, 0))] * 3
        + [pl.BlockSpec((Z, Z), lambda i: (0, 0))] * 3
        + [pl.BlockSpec((1, Z), lambda i: (0, 0))],
        out_specs=pl.BlockSpec((BN, Z), lambda i: (i, 0)),
        out_shape=jax.ShapeDtypeStruct((N_NODES, Z), jnp.float32),
    )(x1, x2, x3, w1, w2, w3, b_row)


def kernel(x, edge_index, edge_attr, W1, b1, F1, bf1, W2, b2, F2, bf2,
           W3, b3, F3, bf3, Wl, bl):
    src = edge_index[0].astype(jnp.int32)
    dst = edge_index[1].astype(jnp.int32)
    zeros_m = jnp.zeros((N_NODES, MSGW), jnp.float32)

    def layer(xc, wfull, b, fw, bf):
        dl = xc.shape[1]
        wa = wfull[:dl]
        wb = wfull[dl:2 * dl]
        wc = wfull[2 * dl:3 * dl]
        wd = wfull[3 * dl:]
        if dl < DPAD:
            wc = jnp.pad(wc, ((0, DPAD - dl), (0, 0)))
        s_tab, d_tab = _make_tables(xc, wa, wb)
        gs, gd = _gather_call()(s_tab, d_tab, src, dst)
        msg = _make_msg(gs, gd, edge_attr, wc, wd, b.reshape(1, Z))
        num = _scatter_call()(msg, dst, zeros_m)
        return _make_update(xc, num[0], num[1], fw[:dl], fw[dl:],
                            bf.reshape(1, Z))

    x1 = layer(x, W1, b1, F1, bf1)
    x2 = layer(x1, W2, b2, F2, bf2)
    x3 = layer(x2, W3, b3, F3, bf3)
    return _make_final(x1, x2, x3, Wl[:Z], Wl[Z:2 * Z], Wl[2 * Z:],
                       bl.reshape(1, Z))


# speedup vs baseline: 3.3104x; 1.0946x over previous
import functools

import jax
import jax.numpy as jnp
from jax import lax
from jax.experimental import pallas as pl
from jax.experimental.pallas import tpu as pltpu
from jax.experimental.pallas import tpu_sc as plsc

N_NODES = 10000
N_EDGES = 320000
DPAD = 128
Z = 96
W_TAB = 128
MSGW = 128
NC, NS = 2, 16
NW = NC * NS
NCH = 5
EC = N_EDGES // NCH
EPC = EC // NW
KB = 80
NBLKC = EPC // KB
RPS = 624
RTAIL = N_NODES - NS * RPS

BN = 2000
BE = 2000


def _bf16_bits(v):
    u = jax.lax.bitcast_convert_type(v, jnp.uint32)
    return (u + jnp.uint32(0x7FFF) + ((u >> 16) & jnp.uint32(1))) >> 16


def _pack2(first, second):
    return (_bf16_bits(second) << 16) | _bf16_bits(first)


def _unpack2(u):
    first = jax.lax.bitcast_convert_type(u << 16, jnp.float32)
    second = jax.lax.bitcast_convert_type(u & jnp.uint32(0xFFFF0000),
                                          jnp.float32)
    return first, second


def _tables_body(x_ref, wa_ref, wb_ref, s_ref, d_ref):
    x = x_ref[...]
    q = jnp.dot(x, wb_ref[...], preferred_element_type=jnp.float32)
    p = jnp.dot(x, wa_ref[...] - wb_ref[...],
                preferred_element_type=jnp.float32)
    dl = x.shape[1]
    hpad = jnp.zeros((x.shape[0], DPAD - Z), jnp.float32)
    if dl < DPAD:
        xpad = jnp.zeros((x.shape[0], DPAD - dl), jnp.float32)
        x = jnp.concatenate([x, xpad], axis=1)
    qh = jnp.concatenate([q, hpad], axis=1)
    ph = jnp.concatenate([p, hpad], axis=1)
    s_ref[...] = _pack2(x, qh)
    d_ref[...] = _pack2(x, ph)


def _make_tables(x, wa, wb):
    dl = x.shape[1]
    return pl.pallas_call(
        _tables_body,
        grid=(N_NODES // BN,),
        in_specs=[
            pl.BlockSpec((BN, dl), lambda i: (i, 0)),
            pl.BlockSpec((dl, Z), lambda i: (0, 0)),
            pl.BlockSpec((dl, Z), lambda i: (0, 0)),
        ],
        out_specs=[pl.BlockSpec((BN, W_TAB), lambda i: (i, 0))] * 2,
        out_shape=[jax.ShapeDtypeStruct((N_NODES, W_TAB), jnp.uint32)] * 2,
    )(x, wa, wb)


def _gather_body(ts, td, src, dst, outs, outd,
                 sidx, didx, srows, drows, sem_s, sem_d):
    wid = lax.axis_index("s") * NC + lax.axis_index("c")

    def blk(i, carry):
        base = wid * EPC + i * KB
        pltpu.sync_copy(src.at[pl.ds(base, KB)], sidx)
        pltpu.sync_copy(dst.at[pl.ds(base, KB)], didx)
        cp1 = pltpu.async_copy(ts.at[sidx], srows, sem_s)
        cp2 = pltpu.async_copy(td.at[didx], drows, sem_d)
        cp1.wait()
        cp2.wait()
        pltpu.sync_copy(srows, outs.at[pl.ds(base, KB)])
        pltpu.sync_copy(drows, outd.at[pl.ds(base, KB)])
        return carry

    lax.fori_loop(0, NBLKC, blk, 0)


def _sc_mesh():
    return plsc.VectorSubcoreMesh(core_axis_name="c", subcore_axis_name="s",
                                  num_cores=NC, num_subcores=NS)


@functools.cache
def _gather_call():
    return pl.kernel(
        _gather_body,
        out_type=[jax.ShapeDtypeStruct((EC, W_TAB), jnp.uint32)] * 2,
        mesh=_sc_mesh(),
        scratch_types=[
            pltpu.VMEM((KB,), jnp.int32),
            pltpu.VMEM((KB,), jnp.int32),
            pltpu.VMEM((KB, W_TAB), jnp.uint32),
            pltpu.VMEM((KB, W_TAB), jnp.uint32),
            pltpu.SemaphoreType.DMA,
            pltpu.SemaphoreType.DMA,
        ],
    )


def _msg_body(gs_ref, gd_ref, ea_ref, wc_ref, wd_ref, b_ref, out_ref):
    xs, qsh = _unpack2(gs_ref[...])
    xd, pdh = _unpack2(gd_ref[...])
    qs = qsh[:, :Z]
    pd = pdh[:, :Z]
    acc = jnp.dot(xs * xd, wc_ref[...], preferred_element_type=jnp.float32)
    acc = acc + jnp.dot(ea_ref[...], wd_ref[...],
                        preferred_element_type=jnp.float32)
    msg = jnp.maximum(acc + qs + pd + b_ref[...], 0.0)
    n = msg.shape[0]
    one = jnp.ones((n, 1), jnp.float32)
    zpad = jnp.zeros((n, MSGW - Z - 1), jnp.float32)
    out_ref[...] = jnp.concatenate([msg, one, zpad], axis=1)


def _make_msg(gs, gd, ea, wc_pad, wd, b_row):
    ea_w = ea.shape[1]
    return pl.pallas_call(
        _msg_body,
        grid=(EC // BE,),
        in_specs=[
            pl.BlockSpec((BE, W_TAB), lambda i: (i, 0)),
            pl.BlockSpec((BE, W_TAB), lambda i: (i, 0)),
            pl.BlockSpec((BE, ea_w), lambda i: (i, 0)),
            pl.BlockSpec((DPAD, Z), lambda i: (0, 0)),
            pl.BlockSpec((ea_w, Z), lambda i: (0, 0)),
            pl.BlockSpec((1, Z), lambda i: (0, 0)),
        ],
        out_specs=pl.BlockSpec((BE, MSGW), lambda i: (i, 0)),
        out_shape=jax.ShapeDtypeStruct((EC, MSGW), jnp.float32),
    )(gs, gd, ea, wc_pad, wd, b_row)


def _scatter_body(m0, m1, m2, m3, m4, dstidx, zz, num_out,
                  idxv, rowsv, accum):
    c = lax.axis_index("c")
    s = lax.axis_index("s")
    wid = s * NC + c
    r0 = s * RPS
    pltpu.sync_copy(zz.at[pl.ds(r0, RPS)], accum.at[pl.ds(r0, RPS)])

    @pl.when(s == 0)
    def _():
        pltpu.sync_copy(zz.at[pl.ds(NS * RPS, RTAIL)],
                        accum.at[pl.ds(NS * RPS, RTAIL)])

    plsc.subcore_barrier()

    for ci, mref in enumerate((m0, m1, m2, m3, m4)):
        def blk(i, carry, ci=ci, mref=mref):
            lbase = wid * EPC + i * KB
            pltpu.sync_copy(dstidx.at[pl.ds(ci * EC + lbase, KB)], idxv)
            pltpu.sync_copy(mref.at[pl.ds(lbase, KB)], rowsv)
            pltpu.sync_copy(rowsv, accum.at[idxv], add=True)
            return carry

        lax.fori_loop(0, NBLKC, blk, 0)
    plsc.subcore_barrier()
    pltpu.sync_copy(accum.at[pl.ds(r0, RPS)], num_out.at[c, pl.ds(r0, RPS)])

    @pl.when(s == 0)
    def _():
        pltpu.sync_copy(accum.at[pl.ds(NS * RPS, RTAIL)],
                        num_out.at[c, pl.ds(NS * RPS, RTAIL)])


@functools.cache
def _scatter_call():
    return pl.kernel(
        _scatter_body,
        out_type=jax.ShapeDtypeStruct((NC, N_NODES, MSGW), jnp.float32),
        mesh=_sc_mesh(),
        scratch_types=[
            pltpu.VMEM((KB,), jnp.int32),
            pltpu.VMEM((KB, MSGW), jnp.float32),
            pltpu.VMEM_SHARED((N_NODES, MSGW), jnp.float32),
        ],
    )


def _update_body(x_ref, n0_ref, n1_ref, fa_ref, fb_ref, bf_ref, out_ref):
    n0 = n0_ref[...]
    n1 = n1_ref[...]
    num = n0[:, :Z] + n1[:, :Z]
    deg = n0[:, Z:Z + 1] + n1[:, Z:Z + 1]
    agg = num / jnp.maximum(deg, 1.0)
    acc = jnp.dot(x_ref[...], fa_ref[...], preferred_element_type=jnp.float32)
    acc = acc + jnp.dot(agg, fb_ref[...], preferred_element_type=jnp.float32)
    out_ref[...] = jnp.maximum(acc + bf_ref[...], 0.0)


def _make_update(x, n0, n1, fa, fb, bf_row):
    dl = x.shape[1]
    return pl.pallas_call(
        _update_body,
        grid=(N_NODES // BN,),
        in_specs=[
            pl.BlockSpec((BN, dl), lambda i: (i, 0)),
            pl.BlockSpec((BN, MSGW), lambda i: (i, 0)),
            pl.BlockSpec((BN, MSGW), lambda i: (i, 0)),
            pl.BlockSpec((dl, Z), lambda i: (0, 0)),
            pl.BlockSpec((Z, Z), lambda i: (0, 0)),
            pl.BlockSpec((1, Z), lambda i: (0, 0)),
        ],
        out_specs=pl.BlockSpec((BN, Z), lambda i: (i, 0)),
        out_shape=jax.ShapeDtypeStruct((N_NODES, Z), jnp.float32),
    )(x, n0, n1, fa, fb, bf_row)


def _final_body(x1_ref, x2_ref, x3_ref, w1_ref, w2_ref, w3_ref, b_ref,
                out_ref):
    acc = jnp.dot(x1_ref[...], w1_ref[...], preferred_element_type=jnp.float32)
    acc = acc + jnp.dot(x2_ref[...], w2_ref[...],
                        preferred_element_type=jnp.float32)
    acc = acc + jnp.dot(x3_ref[...], w3_ref[...],
                        preferred_element_type=jnp.float32)
    out_ref[...] = jnp.maximum(acc + b_ref[...], 0.0)


def _make_final(x1, x2, x3, w1, w2, w3, b_row):
    return pl.pallas_call(
        _final_body,
        grid=(N_NODES // BN,),
        in_specs=[pl.BlockSpec((BN, Z), lambda i: (i, 0))] * 3
        + [pl.BlockSpec((Z, Z), lambda i: (0, 0))] * 3
        + [pl.BlockSpec((1, Z), lambda i: (0, 0))],
        out_specs=pl.BlockSpec((BN, Z), lambda i: (i, 0)),
        out_shape=jax.ShapeDtypeStruct((N_NODES, Z), jnp.float32),
    )(x1, x2, x3, w1, w2, w3, b_row)


def kernel(x, edge_index, edge_attr, W1, b1, F1, bf1, W2, b2, F2, bf2,
           W3, b3, F3, bf3, Wl, bl):
    src = edge_index[0].astype(jnp.int32)
    dst = edge_index[1].astype(jnp.int32)
    zeros_m = jnp.zeros((N_NODES, MSGW), jnp.float32)

    def layer(xc, wfull, b, fw, bf):
        dl = xc.shape[1]
        wa = wfull[:dl]
        wb = wfull[dl:2 * dl]
        wc = wfull[2 * dl:3 * dl]
        wd = wfull[3 * dl:]
        if dl < DPAD:
            wc = jnp.pad(wc, ((0, DPAD - dl), (0, 0)))
        s_tab, d_tab = _make_tables(xc, wa, wb)
        msgs = []
        for ci in range(NCH):
            gs, gd = _gather_call()(s_tab, d_tab,
                                    src[ci * EC:(ci + 1) * EC],
                                    dst[ci * EC:(ci + 1) * EC])
            msgs.append(_make_msg(gs, gd, edge_attr[ci * EC:(ci + 1) * EC],
                                  wc, wd, b.reshape(1, Z)))
        num = _scatter_call()(*msgs, dst, zeros_m)
        return _make_update(xc, num[0], num[1], fw[:dl], fw[dl:],
                            bf.reshape(1, Z))

    x1 = layer(x, W1, b1, F1, bf1)
    x2 = layer(x1, W2, b2, F2, bf2)
    x3 = layer(x2, W3, b3, F3, bf3)
    return _make_final(x1, x2, x3, Wl[:Z], Wl[Z:2 * Z], Wl[2 * Z:],
                       bl.reshape(1, Z))


# speedup vs baseline: 3.4836x; 1.0523x over previous
import functools

import jax
import jax.numpy as jnp
from jax import lax
from jax.experimental import pallas as pl
from jax.experimental.pallas import tpu as pltpu
from jax.experimental.pallas import tpu_sc as plsc

N_NODES = 10000
N_EDGES = 320000
DPAD = 128
Z = 96
W_TAB = 128
MSGW = 128
NC, NS = 2, 16
NW = NC * NS
NCH = 5
EC = N_EDGES // NCH
EPC = EC // NW
KB = 80
NBLKC = EPC // KB
RPS = 624
RTAIL = N_NODES - NS * RPS

BN = 2000
BE = 2000


def _bf16_bits(v):
    u = jax.lax.bitcast_convert_type(v, jnp.uint32)
    return (u + jnp.uint32(0x7FFF) + ((u >> 16) & jnp.uint32(1))) >> 16


def _pack2(first, second):
    return (_bf16_bits(second) << 16) | _bf16_bits(first)


def _unpack2(u):
    first = jax.lax.bitcast_convert_type(u << 16, jnp.float32)
    second = jax.lax.bitcast_convert_type(u & jnp.uint32(0xFFFF0000),
                                          jnp.float32)
    return first, second


def _tables_body(x_ref, wa_ref, wb_ref, s_ref, d_ref):
    x = x_ref[...]
    q = jnp.dot(x, wb_ref[...], preferred_element_type=jnp.float32)
    p = jnp.dot(x, wa_ref[...] - wb_ref[...],
                preferred_element_type=jnp.float32)
    dl = x.shape[1]
    hpad = jnp.zeros((x.shape[0], DPAD - Z), jnp.float32)
    if dl < DPAD:
        xpad = jnp.zeros((x.shape[0], DPAD - dl), jnp.float32)
        x = jnp.concatenate([x, xpad], axis=1)
    qh = jnp.concatenate([q, hpad], axis=1)
    ph = jnp.concatenate([p, hpad], axis=1)
    s_ref[...] = _pack2(x, qh)
    d_ref[...] = _pack2(x, ph)


def _make_tables(x, wa, wb):
    dl = x.shape[1]
    return pl.pallas_call(
        _tables_body,
        grid=(N_NODES // BN,),
        in_specs=[
            pl.BlockSpec((BN, dl), lambda i: (i, 0)),
            pl.BlockSpec((dl, Z), lambda i: (0, 0)),
            pl.BlockSpec((dl, Z), lambda i: (0, 0)),
        ],
        out_specs=[pl.BlockSpec((BN, W_TAB), lambda i: (i, 0))] * 2,
        out_shape=[jax.ShapeDtypeStruct((N_NODES, W_TAB), jnp.uint32)] * 2,
    )(x, wa, wb)


def _gather_body(ts, td, src, dst, outs, outd,
                 sidx0, didx0, sidx1, didx1, sr0, dr0, sr1, dr1,
                 gsem, wsem0, wsem1):
    wid = lax.axis_index("s") * NC + lax.axis_index("c")
    base0 = wid * EPC

    bufs = ((sidx0, didx0, sr0, dr0, wsem0),
            (sidx1, didx1, sr1, dr1, wsem1))

    def do_block(i, buf, wait_prev):
        sidx, didx, sr, dr, wsem = buf
        base = base0 + i * KB

        def drain():
            pltpu.make_async_copy(sr, outs.at[pl.ds(base0, KB)],
                                  wsem).wait()
            pltpu.make_async_copy(dr, outd.at[pl.ds(base0, KB)],
                                  wsem).wait()

        if wait_prev is True:
            drain()
        elif wait_prev is not None:
            pl.when(wait_prev)(drain)

        pltpu.sync_copy(src.at[pl.ds(base, KB)], sidx)
        pltpu.sync_copy(dst.at[pl.ds(base, KB)], didx)
        cp1 = pltpu.async_copy(ts.at[sidx], sr, gsem)
        cp2 = pltpu.async_copy(td.at[didx], dr, gsem)
        cp1.wait()
        cp2.wait()
        pltpu.async_copy(sr, outs.at[pl.ds(base, KB)], wsem)
        pltpu.async_copy(dr, outd.at[pl.ds(base, KB)], wsem)

    def body(k, carry):
        do_block(2 * k, bufs[0], k > 0)
        do_block(2 * k + 1, bufs[1], k > 0)
        return carry

    lax.fori_loop(0, NBLKC // 2, body, 0)
    if NBLKC % 2:
        do_block(NBLKC - 1, bufs[0], NBLKC > 2)
    for bi in (1, 0) if NBLKC % 2 else (0, 1):
        sidx, didx, sr, dr, wsem = bufs[bi]
        pltpu.make_async_copy(sr, outs.at[pl.ds(base0, KB)], wsem).wait()
        pltpu.make_async_copy(dr, outd.at[pl.ds(base0, KB)], wsem).wait()


def _sc_mesh():
    return plsc.VectorSubcoreMesh(core_axis_name="c", subcore_axis_name="s",
                                  num_cores=NC, num_subcores=NS)


@functools.cache
def _gather_call():
    return pl.kernel(
        _gather_body,
        out_type=[jax.ShapeDtypeStruct((EC, W_TAB), jnp.uint32)] * 2,
        mesh=_sc_mesh(),
        scratch_types=[
            pltpu.VMEM((KB,), jnp.int32),
            pltpu.VMEM((KB,), jnp.int32),
            pltpu.VMEM((KB,), jnp.int32),
            pltpu.VMEM((KB,), jnp.int32),
            pltpu.VMEM((KB, W_TAB), jnp.uint32),
            pltpu.VMEM((KB, W_TAB), jnp.uint32),
            pltpu.VMEM((KB, W_TAB), jnp.uint32),
            pltpu.VMEM((KB, W_TAB), jnp.uint32),
            pltpu.SemaphoreType.DMA,
            pltpu.SemaphoreType.DMA,
            pltpu.SemaphoreType.DMA,
        ],
    )


def _msg_body(gs_ref, gd_ref, ea_ref, wc_ref, wd_ref, b_ref, out_ref):
    xs, qsh = _unpack2(gs_ref[...])
    xd, pdh = _unpack2(gd_ref[...])
    qs = qsh[:, :Z]
    pd = pdh[:, :Z]
    acc = jnp.dot(xs * xd, wc_ref[...], preferred_element_type=jnp.float32)
    acc = acc + jnp.dot(ea_ref[...], wd_ref[...],
                        preferred_element_type=jnp.float32)
    msg = jnp.maximum(acc + qs + pd + b_ref[...], 0.0)
    n = msg.shape[0]
    one = jnp.ones((n, 1), jnp.float32)
    zpad = jnp.zeros((n, MSGW - Z - 1), jnp.float32)
    out_ref[...] = jnp.concatenate([msg, one, zpad], axis=1)


def _make_msg(gs, gd, ea, wc_pad, wd, b_row):
    ea_w = ea.shape[1]
    return pl.pallas_call(
        _msg_body,
        grid=(EC // BE,),
        in_specs=[
            pl.BlockSpec((BE, W_TAB), lambda i: (i, 0)),
            pl.BlockSpec((BE, W_TAB), lambda i: (i, 0)),
            pl.BlockSpec((BE, ea_w), lambda i: (i, 0)),
            pl.BlockSpec((DPAD, Z), lambda i: (0, 0)),
            pl.BlockSpec((ea_w, Z), lambda i: (0, 0)),
            pl.BlockSpec((1, Z), lambda i: (0, 0)),
        ],
        out_specs=pl.BlockSpec((BE, MSGW), lambda i: (i, 0)),
        out_shape=jax.ShapeDtypeStruct((EC, MSGW), jnp.float32),
    )(gs, gd, ea, wc_pad, wd, b_row)


def _scatter_body(m0, m1, m2, m3, m4, dstidx, zz, num_out,
                  idxv, rowsv, accum):
    c = lax.axis_index("c")
    s = lax.axis_index("s")
    wid = s * NC + c
    r0 = s * RPS
    pltpu.sync_copy(zz.at[pl.ds(r0, RPS)], accum.at[pl.ds(r0, RPS)])

    @pl.when(s == 0)
    def _():
        pltpu.sync_copy(zz.at[pl.ds(NS * RPS, RTAIL)],
                        accum.at[pl.ds(NS * RPS, RTAIL)])

    plsc.subcore_barrier()

    for ci, mref in enumerate((m0, m1, m2, m3, m4)):
        def blk(i, carry, ci=ci, mref=mref):
            lbase = wid * EPC + i * KB
            pltpu.sync_copy(dstidx.at[pl.ds(ci * EC + lbase, KB)], idxv)
            pltpu.sync_copy(mref.at[pl.ds(lbase, KB)], rowsv)
            pltpu.sync_copy(rowsv, accum.at[idxv], add=True)
            return carry

        lax.fori_loop(0, NBLKC, blk, 0)
    plsc.subcore_barrier()
    pltpu.sync_copy(accum.at[pl.ds(r0, RPS)], num_out.at[c, pl.ds(r0, RPS)])

    @pl.when(s == 0)
    def _():
        pltpu.sync_copy(accum.at[pl.ds(NS * RPS, RTAIL)],
                        num_out.at[c, pl.ds(NS * RPS, RTAIL)])


@functools.cache
def _scatter_call():
    return pl.kernel(
        _scatter_body,
        out_type=jax.ShapeDtypeStruct((NC, N_NODES, MSGW), jnp.float32),
        mesh=_sc_mesh(),
        scratch_types=[
            pltpu.VMEM((KB,), jnp.int32),
            pltpu.VMEM((KB, MSGW), jnp.float32),
            pltpu.VMEM_SHARED((N_NODES, MSGW), jnp.float32),
        ],
    )


def _update_body(x_ref, n0_ref, n1_ref, fa_ref, fb_ref, bf_ref, out_ref):
    n0 = n0_ref[...]
    n1 = n1_ref[...]
    num = n0[:, :Z] + n1[:, :Z]
    deg = n0[:, Z:Z + 1] + n1[:, Z:Z + 1]
    agg = num / jnp.maximum(deg, 1.0)
    acc = jnp.dot(x_ref[...], fa_ref[...], preferred_element_type=jnp.float32)
    acc = acc + jnp.dot(agg, fb_ref[...], preferred_element_type=jnp.float32)
    out_ref[...] = jnp.maximum(acc + bf_ref[...], 0.0)


def _make_update(x, n0, n1, fa, fb, bf_row):
    dl = x.shape[1]
    return pl.pallas_call(
        _update_body,
        grid=(N_NODES // BN,),
        in_specs=[
            pl.BlockSpec((BN, dl), lambda i: (i, 0)),
            pl.BlockSpec((BN, MSGW), lambda i: (i, 0)),
            pl.BlockSpec((BN, MSGW), lambda i: (i, 0)),
            pl.BlockSpec((dl, Z), lambda i: (0, 0)),
            pl.BlockSpec((Z, Z), lambda i: (0, 0)),
            pl.BlockSpec((1, Z), lambda i: (0, 0)),
        ],
        out_specs=pl.BlockSpec((BN, Z), lambda i: (i, 0)),
        out_shape=jax.ShapeDtypeStruct((N_NODES, Z), jnp.float32),
    )(x, n0, n1, fa, fb, bf_row)


def _final_body(x1_ref, x2_ref, x3_ref, w1_ref, w2_ref, w3_ref, b_ref,
                out_ref):
    acc = jnp.dot(x1_ref[...], w1_ref[...], preferred_element_type=jnp.float32)
    acc = acc + jnp.dot(x2_ref[...], w2_ref[...],
                        preferred_element_type=jnp.float32)
    acc = acc + jnp.dot(x3_ref[...], w3_ref[...],
                        preferred_element_type=jnp.float32)
    out_ref[...] = jnp.maximum(acc + b_ref[...], 0.0)


def _make_final(x1, x2, x3, w1, w2, w3, b_row):
    return pl.pallas_call(
        _final_body,
        grid=(N_NODES // BN,),
        in_specs=[pl.BlockSpec((BN, Z), lambda i: (i, 0))] * 3
        + [pl.BlockSpec((Z, Z), lambda i: (0, 0))] * 3
        + [pl.BlockSpec((1, Z), lambda i: (0, 0))],
        out_specs=pl.BlockSpec((BN, Z), lambda i: (i, 0)),
        out_shape=jax.ShapeDtypeStruct((N_NODES, Z), jnp.float32),
    )(x1, x2, x3, w1, w2, w3, b_row)


def kernel(x, edge_index, edge_attr, W1, b1, F1, bf1, W2, b2, F2, bf2,
           W3, b3, F3, bf3, Wl, bl):
    src = edge_index[0].astype(jnp.int32)
    dst = edge_index[1].astype(jnp.int32)
    zeros_m = jnp.zeros((N_NODES, MSGW), jnp.float32)

    def layer(xc, wfull, b, fw, bf):
        dl = xc.shape[1]
        wa = wfull[:dl]
        wb = wfull[dl:2 * dl]
        wc = wfull[2 * dl:3 * dl]
        wd = wfull[3 * dl:]
        if dl < DPAD:
            wc = jnp.pad(wc, ((0, DPAD - dl), (0, 0)))
        s_tab, d_tab = _make_tables(xc, wa, wb)
        msgs = []
        for ci in range(NCH):
            gs, gd = _gather_call()(s_tab, d_tab,
                                    src[ci * EC:(ci + 1) * EC],
                                    dst[ci * EC:(ci + 1) * EC])
            msgs.append(_make_msg(gs, gd, edge_attr[ci * EC:(ci + 1) * EC],
                                  wc, wd, b.reshape(1, Z)))
        num = _scatter_call()(*msgs, dst, zeros_m)
        return _make_update(xc, num[0], num[1], fw[:dl], fw[dl:],
                            bf.reshape(1, Z))

    x1 = layer(x, W1, b1, F1, bf1)
    x2 = layer(x1, W2, b2, F2, bf2)
    x3 = layer(x2, W3, b3, F3, bf3)
    return _make_final(x1, x2, x3, Wl[:Z], Wl[Z:2 * Z], Wl[2 * Z:],
                       bl.reshape(1, Z))


# speedup vs baseline: 3.9629x; 1.1376x over previous
import functools

import jax
import jax.numpy as jnp
from jax import lax
from jax.experimental import pallas as pl
from jax.experimental.pallas import tpu as pltpu
from jax.experimental.pallas import tpu_sc as plsc

N_NODES = 10000
N_EDGES = 320000
DPAD = 128
Z = 96
W_TAB = 128
MSGW = 128
NC, NS = 2, 16
NW = NC * NS
NCH = 5
EC = N_EDGES // NCH
EPC = EC // NW
KB = 80
NBLKC = EPC // KB
RPS = 624
RTAIL = N_NODES - NS * RPS

BN = 2000
BE = 2000


def _bf16_bits(v):
    u = jax.lax.bitcast_convert_type(v, jnp.uint32)
    return (u + jnp.uint32(0x7FFF) + ((u >> 16) & jnp.uint32(1))) >> 16


def _pack2(first, second):
    return (_bf16_bits(second) << 16) | _bf16_bits(first)


def _unpack2(u):
    first = jax.lax.bitcast_convert_type(u << 16, jnp.float32)
    second = jax.lax.bitcast_convert_type(u & jnp.uint32(0xFFFF0000),
                                          jnp.float32)
    return first, second


def _tables_body(x_ref, wa_ref, wb_ref, s_ref, d_ref):
    x = x_ref[...]
    q = jnp.dot(x, wb_ref[...], preferred_element_type=jnp.float32)
    p = jnp.dot(x, wa_ref[...] - wb_ref[...],
                preferred_element_type=jnp.float32)
    dl = x.shape[1]
    hpad = jnp.zeros((x.shape[0], DPAD - Z), jnp.float32)
    if dl < DPAD:
        xpad = jnp.zeros((x.shape[0], DPAD - dl), jnp.float32)
        x = jnp.concatenate([x, xpad], axis=1)
    qh = jnp.concatenate([q, hpad], axis=1)
    ph = jnp.concatenate([p, hpad], axis=1)
    s_ref[...] = _pack2(x, qh)
    d_ref[...] = _pack2(x, ph)


def _make_tables(x, wa, wb):
    dl = x.shape[1]
    return pl.pallas_call(
        _tables_body,
        grid=(N_NODES // BN,),
        in_specs=[
            pl.BlockSpec((BN, dl), lambda i: (i, 0)),
            pl.BlockSpec((dl, Z), lambda i: (0, 0)),
            pl.BlockSpec((dl, Z), lambda i: (0, 0)),
        ],
        out_specs=[pl.BlockSpec((BN, W_TAB), lambda i: (i, 0))] * 2,
        out_shape=[jax.ShapeDtypeStruct((N_NODES, W_TAB), jnp.uint32)] * 2,
    )(x, wa, wb)


def _gather_body(ts, td, src, dst, outs, outd,
                 sidx0, didx0, sidx1, didx1, sr0, dr0, sr1, dr1,
                 gsem, wsem0, wsem1):
    wid = lax.axis_index("s") * NC + lax.axis_index("c")
    base0 = wid * EPC

    bufs = ((sidx0, didx0, sr0, dr0, wsem0),
            (sidx1, didx1, sr1, dr1, wsem1))

    def do_block(i, buf, wait_prev):
        sidx, didx, sr, dr, wsem = buf
        base = base0 + i * KB

        def drain():
            pltpu.make_async_copy(sr, outs.at[pl.ds(base0, KB)],
                                  wsem).wait()
            pltpu.make_async_copy(dr, outd.at[pl.ds(base0, KB)],
                                  wsem).wait()

        if wait_prev is True:
            drain()
        elif wait_prev is not None:
            pl.when(wait_prev)(drain)

        pltpu.sync_copy(src.at[pl.ds(base, KB)], sidx)
        pltpu.sync_copy(dst.at[pl.ds(base, KB)], didx)
        cp1 = pltpu.async_copy(ts.at[sidx], sr, gsem)
        cp2 = pltpu.async_copy(td.at[didx], dr, gsem)
        cp1.wait()
        cp2.wait()
        pltpu.async_copy(sr, outs.at[pl.ds(base, KB)], wsem)
        pltpu.async_copy(dr, outd.at[pl.ds(base, KB)], wsem)

    def body(k, carry):
        do_block(2 * k, bufs[0], k > 0)
        do_block(2 * k + 1, bufs[1], k > 0)
        return carry

    lax.fori_loop(0, NBLKC // 2, body, 0)
    if NBLKC % 2:
        do_block(NBLKC - 1, bufs[0], NBLKC > 2)
    for bi in (1, 0) if NBLKC % 2 else (0, 1):
        sidx, didx, sr, dr, wsem = bufs[bi]
        pltpu.make_async_copy(sr, outs.at[pl.ds(base0, KB)], wsem).wait()
        pltpu.make_async_copy(dr, outd.at[pl.ds(base0, KB)], wsem).wait()


def _sc_mesh():
    return plsc.VectorSubcoreMesh(core_axis_name="c", subcore_axis_name="s",
                                  num_cores=NC, num_subcores=NS)


@functools.cache
def _gather_call():
    return pl.kernel(
        _gather_body,
        out_type=[jax.ShapeDtypeStruct((EC, W_TAB), jnp.uint32)] * 2,
        mesh=_sc_mesh(),
        scratch_types=[
            pltpu.VMEM((KB,), jnp.int32),
            pltpu.VMEM((KB,), jnp.int32),
            pltpu.VMEM((KB,), jnp.int32),
            pltpu.VMEM((KB,), jnp.int32),
            pltpu.VMEM((KB, W_TAB), jnp.uint32),
            pltpu.VMEM((KB, W_TAB), jnp.uint32),
            pltpu.VMEM((KB, W_TAB), jnp.uint32),
            pltpu.VMEM((KB, W_TAB), jnp.uint32),
            pltpu.SemaphoreType.DMA,
            pltpu.SemaphoreType.DMA,
            pltpu.SemaphoreType.DMA,
        ],
    )


def _msg_body(gs_ref, gd_ref, ea_ref, wc_ref, wd_ref, b_ref, out_ref):
    xs, qsh = _unpack2(gs_ref[...])
    xd, pdh = _unpack2(gd_ref[...])
    qs = qsh[:, :Z]
    pd = pdh[:, :Z]
    acc = jnp.dot(xs * xd, wc_ref[...], preferred_element_type=jnp.float32)
    acc = acc + jnp.dot(ea_ref[...], wd_ref[...],
                        preferred_element_type=jnp.float32)
    msg = jnp.maximum(acc + qs + pd + b_ref[...], 0.0)
    n = msg.shape[0]
    one = jnp.ones((n, 1), jnp.float32)
    zpad = jnp.zeros((n, MSGW - Z - 1), jnp.float32)
    out_ref[...] = jnp.concatenate([msg, one, zpad], axis=1)


def _make_msg(gs, gd, ea, wc_pad, wd, b_row):
    ea_w = ea.shape[1]
    return pl.pallas_call(
        _msg_body,
        grid=(EC // BE,),
        in_specs=[
            pl.BlockSpec((BE, W_TAB), lambda i: (i, 0)),
            pl.BlockSpec((BE, W_TAB), lambda i: (i, 0)),
            pl.BlockSpec((BE, ea_w), lambda i: (i, 0)),
            pl.BlockSpec((DPAD, Z), lambda i: (0, 0)),
            pl.BlockSpec((ea_w, Z), lambda i: (0, 0)),
            pl.BlockSpec((1, Z), lambda i: (0, 0)),
        ],
        out_specs=pl.BlockSpec((BE, MSGW), lambda i: (i, 0)),
        out_shape=jax.ShapeDtypeStruct((EC, MSGW), jnp.float32),
    )(gs, gd, ea, wc_pad, wd, b_row)


def _scatter_body(m0, m1, m2, m3, m4, dstidx, zz, num_out,
                  idx0, rows0, idx1, rows1, lsem0, lsem1, accum):
    c = lax.axis_index("c")
    s = lax.axis_index("s")
    wid = s * NC + c
    r0 = s * RPS
    pltpu.sync_copy(zz.at[pl.ds(r0, RPS)], accum.at[pl.ds(r0, RPS)])

    @pl.when(s == 0)
    def _():
        pltpu.sync_copy(zz.at[pl.ds(NS * RPS, RTAIL)],
                        accum.at[pl.ds(NS * RPS, RTAIL)])

    plsc.subcore_barrier()

    for ci, mref in enumerate((m0, m1, m2, m3, m4)):
        def load(i, idxv, rowsv, lsem):
            lbase = wid * EPC + i * KB
            pltpu.async_copy(dstidx.at[pl.ds(ci * EC + lbase, KB)], idxv,
                             lsem)
            pltpu.async_copy(mref.at[pl.ds(lbase, KB)], rowsv, lsem)

        def wait_load(idxv, rowsv, lsem):
            pltpu.make_async_copy(dstidx.at[pl.ds(wid * EPC, KB)], idxv,
                                  lsem).wait()
            pltpu.make_async_copy(mref.at[pl.ds(wid * EPC, KB)], rowsv,
                                  lsem).wait()

        load(0, idx0, rows0, lsem0)

        def blk(k, carry):
            wait_load(idx0, rows0, lsem0)
            load(2 * k + 1, idx1, rows1, lsem1)
            pltpu.sync_copy(rows0, accum.at[idx0], add=True)
            wait_load(idx1, rows1, lsem1)
            load(2 * k + 2, idx0, rows0, lsem0)
            pltpu.sync_copy(rows1, accum.at[idx1], add=True)
            return carry

        lax.fori_loop(0, NBLKC // 2, blk, 0)
        wait_load(idx0, rows0, lsem0)
        pltpu.sync_copy(rows0, accum.at[idx0], add=True)
    plsc.subcore_barrier()
    pltpu.sync_copy(accum.at[pl.ds(r0, RPS)], num_out.at[c, pl.ds(r0, RPS)])

    @pl.when(s == 0)
    def _():
        pltpu.sync_copy(accum.at[pl.ds(NS * RPS, RTAIL)],
                        num_out.at[c, pl.ds(NS * RPS, RTAIL)])


@functools.cache
def _scatter_call():
    return pl.kernel(
        _scatter_body,
        out_type=jax.ShapeDtypeStruct((NC, N_NODES, MSGW), jnp.float32),
        mesh=_sc_mesh(),
        scratch_types=[
            pltpu.VMEM((KB,), jnp.int32),
            pltpu.VMEM((KB, MSGW), jnp.float32),
            pltpu.VMEM((KB,), jnp.int32),
            pltpu.VMEM((KB, MSGW), jnp.float32),
            pltpu.SemaphoreType.DMA,
            pltpu.SemaphoreType.DMA,
            pltpu.VMEM_SHARED((N_NODES, MSGW), jnp.float32),
        ],
    )


def _update_body(x_ref, n0_ref, n1_ref, fa_ref, fb_ref, bf_ref, out_ref):
    n0 = n0_ref[...]
    n1 = n1_ref[...]
    num = n0[:, :Z] + n1[:, :Z]
    deg = n0[:, Z:Z + 1] + n1[:, Z:Z + 1]
    agg = num / jnp.maximum(deg, 1.0)
    acc = jnp.dot(x_ref[...], fa_ref[...], preferred_element_type=jnp.float32)
    acc = acc + jnp.dot(agg, fb_ref[...], preferred_element_type=jnp.float32)
    out_ref[...] = jnp.maximum(acc + bf_ref[...], 0.0)


def _make_update(x, n0, n1, fa, fb, bf_row):
    dl = x.shape[1]
    return pl.pallas_call(
        _update_body,
        grid=(N_NODES // BN,),
        in_specs=[
            pl.BlockSpec((BN, dl), lambda i: (i, 0)),
            pl.BlockSpec((BN, MSGW), lambda i: (i, 0)),
            pl.BlockSpec((BN, MSGW), lambda i: (i, 0)),
            pl.BlockSpec((dl, Z), lambda i: (0, 0)),
            pl.BlockSpec((Z, Z), lambda i: (0, 0)),
            pl.BlockSpec((1, Z), lambda i: (0, 0)),
        ],
        out_specs=pl.BlockSpec((BN, Z), lambda i: (i, 0)),
        out_shape=jax.ShapeDtypeStruct((N_NODES, Z), jnp.float32),
    )(x, n0, n1, fa, fb, bf_row)


def _final_body(x1_ref, x2_ref, x3_ref, w1_ref, w2_ref, w3_ref, b_ref,
                out_ref):
    acc = jnp.dot(x1_ref[...], w1_ref[...], preferred_element_type=jnp.float32)
    acc = acc + jnp.dot(x2_ref[...], w2_ref[...],
                        preferred_element_type=jnp.float32)
    acc = acc + jnp.dot(x3_ref[...], w3_ref[...],
                        preferred_element_type=jnp.float32)
    out_ref[...] = jnp.maximum(acc + b_ref[...], 0.0)


def _make_final(x1, x2, x3, w1, w2, w3, b_row):
    return pl.pallas_call(
        _final_body,
        grid=(N_NODES // BN,),
        in_specs=[pl.BlockSpec((BN, Z), lambda i: (i, 0))] * 3
        + [pl.BlockSpec((Z, Z), lambda i: (0, 0))] * 3
        + [pl.BlockSpec((1, Z), lambda i: (0, 0))],
        out_specs=pl.BlockSpec((BN, Z), lambda i: (i, 0)),
        out_shape=jax.ShapeDtypeStruct((N_NODES, Z), jnp.float32),
    )(x1, x2, x3, w1, w2, w3, b_row)


def kernel(x, edge_index, edge_attr, W1, b1, F1, bf1, W2, b2, F2, bf2,
           W3, b3, F3, bf3, Wl, bl):
    src = edge_index[0].astype(jnp.int32)
    dst = edge_index[1].astype(jnp.int32)
    zeros_m = jnp.zeros((N_NODES, MSGW), jnp.float32)

    def layer(xc, wfull, b, fw, bf):
        dl = xc.shape[1]
        wa = wfull[:dl]
        wb = wfull[dl:2 * dl]
        wc = wfull[2 * dl:3 * dl]
        wd = wfull[3 * dl:]
        if dl < DPAD:
            wc = jnp.pad(wc, ((0, DPAD - dl), (0, 0)))
        s_tab, d_tab = _make_tables(xc, wa, wb)
        msgs = []
        for ci in range(NCH):
            gs, gd = _gather_call()(s_tab, d_tab,
                                    src[ci * EC:(ci + 1) * EC],
                                    dst[ci * EC:(ci + 1) * EC])
            msgs.append(_make_msg(gs, gd, edge_attr[ci * EC:(ci + 1) * EC],
                                  wc, wd, b.reshape(1, Z)))
        num = _scatter_call()(*msgs, dst, zeros_m)
        return _make_update(xc, num[0], num[1], fw[:dl], fw[dl:],
                            bf.reshape(1, Z))

    x1 = layer(x, W1, b1, F1, bf1)
    x2 = layer(x1, W2, b2, F2, bf2)
    x3 = layer(x2, W3, b3, F3, bf3)
    return _make_final(x1, x2, x3, Wl[:Z], Wl[Z:2 * Z], Wl[2 * Z:],
                       bl.reshape(1, Z))


# speedup vs baseline: 4.2628x; 1.0757x over previous
import functools

import jax
import jax.numpy as jnp
from jax import lax
from jax.experimental import pallas as pl
from jax.experimental.pallas import tpu as pltpu
from jax.experimental.pallas import tpu_sc as plsc

N_NODES = 10000
N_EDGES = 320000
DPAD = 128
Z = 96
W_TAB = 128
MSGW = 128
NC, NS = 2, 16
NW = NC * NS
NCH = 5
EC = N_EDGES // NCH
EPC = EC // NW
KB = 80
NBLKC = EPC // KB
RPS = 624
RTAIL = N_NODES - NS * RPS

BN = 2000
BE = 2000


def _bf16_bits(v):
    u = jax.lax.bitcast_convert_type(v, jnp.uint32)
    return (u + jnp.uint32(0x7FFF) + ((u >> 16) & jnp.uint32(1))) >> 16


def _pack2(first, second):
    return (_bf16_bits(second) << 16) | _bf16_bits(first)


def _unpack2(u):
    first = jax.lax.bitcast_convert_type(u << 16, jnp.float32)
    second = jax.lax.bitcast_convert_type(u & jnp.uint32(0xFFFF0000),
                                          jnp.float32)
    return first, second


def _tables_body(x_ref, wa_ref, wb_ref, s_ref, d_ref):
    x = x_ref[...]
    q = jnp.dot(x, wb_ref[...], preferred_element_type=jnp.float32)
    p = jnp.dot(x, wa_ref[...] - wb_ref[...],
                preferred_element_type=jnp.float32)
    dl = x.shape[1]
    hpad = jnp.zeros((x.shape[0], DPAD - Z), jnp.float32)
    if dl < DPAD:
        xpad = jnp.zeros((x.shape[0], DPAD - dl), jnp.float32)
        x = jnp.concatenate([x, xpad], axis=1)
    qh = jnp.concatenate([q, hpad], axis=1)
    ph = jnp.concatenate([p, hpad], axis=1)
    s_ref[...] = _pack2(x, qh)
    d_ref[...] = _pack2(x, ph)


def _make_tables(x, wa, wb):
    dl = x.shape[1]
    return pl.pallas_call(
        _tables_body,
        grid=(N_NODES // BN,),
        in_specs=[
            pl.BlockSpec((BN, dl), lambda i: (i, 0)),
            pl.BlockSpec((dl, Z), lambda i: (0, 0)),
            pl.BlockSpec((dl, Z), lambda i: (0, 0)),
        ],
        out_specs=[pl.BlockSpec((BN, W_TAB), lambda i: (i, 0))] * 2,
        out_shape=[jax.ShapeDtypeStruct((N_NODES, W_TAB), jnp.uint32)] * 2,
    )(x, wa, wb)


def _gather_body(ts, td, src, dst, outs, outd,
                 sidx_all, didx_all, sr0, dr0, sr1, dr1,
                 gsem, wsem0, wsem1):
    wid = lax.axis_index("s") * NC + lax.axis_index("c")
    base0 = wid * EPC

    pltpu.sync_copy(src.at[pl.ds(base0, EPC)], sidx_all)
    pltpu.sync_copy(dst.at[pl.ds(base0, EPC)], didx_all)

    bufs = ((sr0, dr0, wsem0), (sr1, dr1, wsem1))

    def do_block(i, buf, wait_prev):
        sr, dr, wsem = buf
        base = base0 + i * KB

        def drain():
            pltpu.make_async_copy(sr, outs.at[pl.ds(base0, KB)],
                                  wsem).wait()
            pltpu.make_async_copy(dr, outd.at[pl.ds(base0, KB)],
                                  wsem).wait()

        if wait_prev is True:
            drain()
        elif wait_prev is not None:
            pl.when(wait_prev)(drain)

        cp1 = pltpu.async_copy(ts.at[sidx_all.at[pl.ds(i * KB, KB)]], sr,
                               gsem)
        cp2 = pltpu.async_copy(td.at[didx_all.at[pl.ds(i * KB, KB)]], dr,
                               gsem)
        cp1.wait()
        cp2.wait()
        pltpu.async_copy(sr, outs.at[pl.ds(base, KB)], wsem)
        pltpu.async_copy(dr, outd.at[pl.ds(base, KB)], wsem)

    def body(k, carry):
        do_block(2 * k, bufs[0], k > 0)
        do_block(2 * k + 1, bufs[1], k > 0)
        return carry

    lax.fori_loop(0, NBLKC // 2, body, 0)
    if NBLKC % 2:
        do_block(NBLKC - 1, bufs[0], NBLKC > 2)
    for bi in (1, 0) if NBLKC % 2 else (0, 1):
        sr, dr, wsem = bufs[bi]
        pltpu.make_async_copy(sr, outs.at[pl.ds(base0, KB)], wsem).wait()
        pltpu.make_async_copy(dr, outd.at[pl.ds(base0, KB)], wsem).wait()


def _sc_mesh():
    return plsc.VectorSubcoreMesh(core_axis_name="c", subcore_axis_name="s",
                                  num_cores=NC, num_subcores=NS)


@functools.cache
def _gather_call():
    return pl.kernel(
        _gather_body,
        out_type=[jax.ShapeDtypeStruct((EC, W_TAB), jnp.uint32)] * 2,
        mesh=_sc_mesh(),
        scratch_types=[
            pltpu.VMEM((EPC,), jnp.int32),
            pltpu.VMEM((EPC,), jnp.int32),
            pltpu.VMEM((KB, W_TAB), jnp.uint32),
            pltpu.VMEM((KB, W_TAB), jnp.uint32),
            pltpu.VMEM((KB, W_TAB), jnp.uint32),
            pltpu.VMEM((KB, W_TAB), jnp.uint32),
            pltpu.SemaphoreType.DMA,
            pltpu.SemaphoreType.DMA,
            pltpu.SemaphoreType.DMA,
        ],
    )


def _msg_body(gs_ref, gd_ref, ea_ref, wc_ref, wd_ref, b_ref, out_ref):
    xs, qsh = _unpack2(gs_ref[...])
    xd, pdh = _unpack2(gd_ref[...])
    qs = qsh[:, :Z]
    pd = pdh[:, :Z]
    acc = jnp.dot(xs * xd, wc_ref[...], preferred_element_type=jnp.float32)
    acc = acc + jnp.dot(ea_ref[...], wd_ref[...],
                        preferred_element_type=jnp.float32)
    msg = jnp.maximum(acc + qs + pd + b_ref[...], 0.0)
    n = msg.shape[0]
    one = jnp.ones((n, 1), jnp.float32)
    zpad = jnp.zeros((n, MSGW - Z - 1), jnp.float32)
    out_ref[...] = jnp.concatenate([msg, one, zpad], axis=1)


def _make_msg(gs, gd, ea, wc_pad, wd, b_row):
    ea_w = ea.shape[1]
    return pl.pallas_call(
        _msg_body,
        grid=(EC // BE,),
        in_specs=[
            pl.BlockSpec((BE, W_TAB), lambda i: (i, 0)),
            pl.BlockSpec((BE, W_TAB), lambda i: (i, 0)),
            pl.BlockSpec((BE, ea_w), lambda i: (i, 0)),
            pl.BlockSpec((DPAD, Z), lambda i: (0, 0)),
            pl.BlockSpec((ea_w, Z), lambda i: (0, 0)),
            pl.BlockSpec((1, Z), lambda i: (0, 0)),
        ],
        out_specs=pl.BlockSpec((BE, MSGW), lambda i: (i, 0)),
        out_shape=jax.ShapeDtypeStruct((EC, MSGW), jnp.float32),
    )(gs, gd, ea, wc_pad, wd, b_row)


def _scatter_body(m0, m1, m2, m3, m4, dstidx, zz, num_out,
                  idx0, rows0, idx1, rows1, lsem0, lsem1, accum):
    c = lax.axis_index("c")
    s = lax.axis_index("s")
    wid = s * NC + c
    r0 = s * RPS
    pltpu.sync_copy(zz.at[pl.ds(r0, RPS)], accum.at[pl.ds(r0, RPS)])

    @pl.when(s == 0)
    def _():
        pltpu.sync_copy(zz.at[pl.ds(NS * RPS, RTAIL)],
                        accum.at[pl.ds(NS * RPS, RTAIL)])

    plsc.subcore_barrier()

    for ci, mref in enumerate((m0, m1, m2, m3, m4)):
        def load(i, idxv, rowsv, lsem):
            lbase = wid * EPC + i * KB
            pltpu.async_copy(dstidx.at[pl.ds(ci * EC + lbase, KB)], idxv,
                             lsem)
            pltpu.async_copy(mref.at[pl.ds(lbase, KB)], rowsv, lsem)

        def wait_load(idxv, rowsv, lsem):
            pltpu.make_async_copy(dstidx.at[pl.ds(wid * EPC, KB)], idxv,
                                  lsem).wait()
            pltpu.make_async_copy(mref.at[pl.ds(wid * EPC, KB)], rowsv,
                                  lsem).wait()

        load(0, idx0, rows0, lsem0)

        def blk(k, carry):
            wait_load(idx0, rows0, lsem0)
            load(2 * k + 1, idx1, rows1, lsem1)
            pltpu.sync_copy(rows0, accum.at[idx0], add=True)
            wait_load(idx1, rows1, lsem1)
            load(2 * k + 2, idx0, rows0, lsem0)
            pltpu.sync_copy(rows1, accum.at[idx1], add=True)
            return carry

        lax.fori_loop(0, NBLKC // 2, blk, 0)
        wait_load(idx0, rows0, lsem0)
        pltpu.sync_copy(rows0, accum.at[idx0], add=True)
    plsc.subcore_barrier()
    pltpu.sync_copy(accum.at[pl.ds(r0, RPS)], num_out.at[c, pl.ds(r0, RPS)])

    @pl.when(s == 0)
    def _():
        pltpu.sync_copy(accum.at[pl.ds(NS * RPS, RTAIL)],
                        num_out.at[c, pl.ds(NS * RPS, RTAIL)])


@functools.cache
def _scatter_call():
    return pl.kernel(
        _scatter_body,
        out_type=jax.ShapeDtypeStruct((NC, N_NODES, MSGW), jnp.float32),
        mesh=_sc_mesh(),
        scratch_types=[
            pltpu.VMEM((KB,), jnp.int32),
            pltpu.VMEM((KB, MSGW), jnp.float32),
            pltpu.VMEM((KB,), jnp.int32),
            pltpu.VMEM((KB, MSGW), jnp.float32),
            pltpu.SemaphoreType.DMA,
            pltpu.SemaphoreType.DMA,
            pltpu.VMEM_SHARED((N_NODES, MSGW), jnp.float32),
        ],
    )


def _update_body(x_ref, n0_ref, n1_ref, fa_ref, fb_ref, bf_ref, out_ref):
    n0 = n0_ref[...]
    n1 = n1_ref[...]
    num = n0[:, :Z] + n1[:, :Z]
    deg = n0[:, Z:Z + 1] + n1[:, Z:Z + 1]
    agg = num / jnp.maximum(deg, 1.0)
    acc = jnp.dot(x_ref[...], fa_ref[...], preferred_element_type=jnp.float32)
    acc = acc + jnp.dot(agg, fb_ref[...], preferred_element_type=jnp.float32)
    out_ref[...] = jnp.maximum(acc + bf_ref[...], 0.0)


def _make_update(x, n0, n1, fa, fb, bf_row):
    dl = x.shape[1]
    return pl.pallas_call(
        _update_body,
        grid=(N_NODES // BN,),
        in_specs=[
            pl.BlockSpec((BN, dl), lambda i: (i, 0)),
            pl.BlockSpec((BN, MSGW), lambda i: (i, 0)),
            pl.BlockSpec((BN, MSGW), lambda i: (i, 0)),
            pl.BlockSpec((dl, Z), lambda i: (0, 0)),
            pl.BlockSpec((Z, Z), lambda i: (0, 0)),
            pl.BlockSpec((1, Z), lambda i: (0, 0)),
        ],
        out_specs=pl.BlockSpec((BN, Z), lambda i: (i, 0)),
        out_shape=jax.ShapeDtypeStruct((N_NODES, Z), jnp.float32),
    )(x, n0, n1, fa, fb, bf_row)


def _final_body(x1_ref, x2_ref, x3_ref, w1_ref, w2_ref, w3_ref, b_ref,
                out_ref):
    acc = jnp.dot(x1_ref[...], w1_ref[...], preferred_element_type=jnp.float32)
    acc = acc + jnp.dot(x2_ref[...], w2_ref[...],
                        preferred_element_type=jnp.float32)
    acc = acc + jnp.dot(x3_ref[...], w3_ref[...],
                        preferred_element_type=jnp.float32)
    out_ref[...] = jnp.maximum(acc + b_ref[...], 0.0)


def _make_final(x1, x2, x3, w1, w2, w3, b_row):
    return pl.pallas_call(
        _final_body,
        grid=(N_NODES // BN,),
        in_specs=[pl.BlockSpec((BN, Z), lambda i: (i, 0))] * 3
        + [pl.BlockSpec((Z, Z), lambda i: (0, 0))] * 3
        + [pl.BlockSpec((1, Z), lambda i: (0, 0))],
        out_specs=pl.BlockSpec((BN, Z), lambda i: (i, 0)),
        out_shape=jax.ShapeDtypeStruct((N_NODES, Z), jnp.float32),
    )(x1, x2, x3, w1, w2, w3, b_row)


def kernel(x, edge_index, edge_attr, W1, b1, F1, bf1, W2, b2, F2, bf2,
           W3, b3, F3, bf3, Wl, bl):
    src = edge_index[0].astype(jnp.int32)
    dst = edge_index[1].astype(jnp.int32)
    zeros_m = jnp.zeros((N_NODES, MSGW), jnp.float32)

    def layer(xc, wfull, b, fw, bf):
        dl = xc.shape[1]
        wa = wfull[:dl]
        wb = wfull[dl:2 * dl]
        wc = wfull[2 * dl:3 * dl]
        wd = wfull[3 * dl:]
        if dl < DPAD:
            wc = jnp.pad(wc, ((0, DPAD - dl), (0, 0)))
        s_tab, d_tab = _make_tables(xc, wa, wb)
        msgs = []
        for ci in range(NCH):
            gs, gd = _gather_call()(s_tab, d_tab,
                                    src[ci * EC:(ci + 1) * EC],
                                    dst[ci * EC:(ci + 1) * EC])
            msgs.append(_make_msg(gs, gd, edge_attr[ci * EC:(ci + 1) * EC],
                                  wc, wd, b.reshape(1, Z)))
        num = _scatter_call()(*msgs, dst, zeros_m)
        return _make_update(xc, num[0], num[1], fw[:dl], fw[dl:],
                            bf.reshape(1, Z))

    x1 = layer(x, W1, b1, F1, bf1)
    x2 = layer(x1, W2, b2, F2, bf2)
    x3 = layer(x2, W3, b3, F3, bf3)
    return _make_final(x1, x2, x3, Wl[:Z], Wl[Z:2 * Z], Wl[2 * Z:],
                       bl.reshape(1, Z))
